# baseline (speedup 1.0000x reference)
import jax
import jax.numpy as jnp
from jax import lax
from jax.experimental import pallas as pl
from jax.experimental.pallas import tpu as pltpu
from jax.experimental.pallas import tpu_sc as plsc

NUM_CHANNELS = 8
VOCAB_P1 = 1001
EMBED = 128

NC, NS, L = 2, 16, 16
NW = NC * NS

CH = 128
NBUF = 2
W = 1

TAB_PAD = 8064


def _body(tab_hbm, idx_hbm, out_hbm, tab_sp, idx_all,
          r0, r1, gs0, gs1, ws0, ws1):
    rows = [r0, r1]
    gsem = [gs0, gs1]
    wsem = [ws0, ws1]
    n_ch = idx_all.shape[0]
    sid = lax.axis_index("s")
    wid = sid * NC + lax.axis_index("c")
    row_base = wid * n_ch

    seg = TAB_PAD // NS
    pltpu.sync_copy(tab_hbm.at[pl.ds(sid * seg, seg)],
                    tab_sp.at[pl.ds(sid * seg, seg)])

    pltpu.sync_copy(idx_hbm.at[wid], idx_all)
    offs = lax.rem(lax.iota(jnp.int32, L), NUM_CHANNELS) * VOCAB_P1

    def add_off(g):
        r = idx_all.at[g]
        for j in range(CH // L):
            r[pl.ds(j * L, L)] = r[pl.ds(j * L, L)] + offs

    plsc.subcore_barrier()

    def g_start(g, b):
        pltpu.make_async_copy(tab_sp.at[idx_all.at[g]], rows[b], gsem[b]).start()

    def g_wait(g, b):
        pltpu.make_async_copy(tab_sp.at[idx_all.at[g]], rows[b], gsem[b]).wait()

    def w_start(g, b):
        base = pl.multiple_of((row_base + g) * CH, CH)
        pltpu.make_async_copy(rows[b], out_hbm.at[pl.ds(base, CH)], wsem[b]).start()

    def w_wait(g, b):
        base = pl.multiple_of((row_base + g) * CH, CH)
        pltpu.make_async_copy(rows[b], out_hbm.at[pl.ds(base, CH)], wsem[b]).wait()

    for i in range(NBUF):
        add_off(i)
        g_start(i, i)
        if i >= W:
            g_wait(i - W, i - W)
            w_start(i - W, i - W)

    def outer(q, c):
        g0 = q * NBUF
        for b in range(NBUF):
            g = g0 + b
            w_wait(g - NBUF, b)
            add_off(g)
            g_start(g, b)
            b2 = (b - W) % NBUF
            g_wait(g - W, b2)
            w_start(g - W, b2)
        return c

    lax.fori_loop(1, n_ch // NBUF, outer, 0)

    for i in range(W):
        g = n_ch - W + i
        b = g % NBUF
        g_wait(g, b)
        w_start(g, b)
    for b in range(NBUF):
        w_wait(n_ch - NBUF + b, b)


def kernel(x, tables):
    B, T, C = x.shape
    n = B * T * C
    n_ch = n // CH // NW
    flat_idx = x.reshape(NW, n_ch, CH).astype(jnp.int32)
    flat_tab = tables.reshape(-1, EMBED)
    flat_tab = jnp.concatenate(
        [flat_tab, jnp.zeros((TAB_PAD - flat_tab.shape[0], EMBED), jnp.float32)])

    k = pl.kernel(
        _body,
        out_type=jax.ShapeDtypeStruct((n, EMBED), jnp.float32),
        mesh=plsc.VectorSubcoreMesh(core_axis_name="c", subcore_axis_name="s"),
        scratch_types=(
            [pltpu.VMEM_SHARED((TAB_PAD, EMBED), jnp.float32)]
            + [pltpu.VMEM((n_ch, CH), jnp.int32)]
            + [pltpu.VMEM((CH, EMBED), jnp.float32)] * NBUF
            + [pltpu.SemaphoreType.DMA] * (2 * NBUF)
        ),
    )
    out = k(flat_tab, flat_idx)
    return out.reshape(B, T, C, EMBED)


# speedup vs baseline: 1.0309x; 1.0309x over previous
import jax
import jax.numpy as jnp
from jax import lax
from jax.experimental import pallas as pl
from jax.experimental.pallas import tpu as pltpu
from jax.experimental.pallas import tpu_sc as plsc

NUM_CHANNELS = 8
VOCAB_P1 = 1001
EMBED = 128

NC, NS = 2, 16
NW = NC * NS

CH = 64
NBUF = 5
W = 3

TAB_PAD = 8064


def _body(tab_hbm, idx_hbm, out_hbm, tab_sp, idx_all,
          r0, r1, r2, r3, r4, gs0, gs1, gs2, gs3, gs4, ws0, ws1, ws2, ws3, ws4):
    rows = [r0, r1, r2, r3, r4]
    gsem = [gs0, gs1, gs2, gs3, gs4]
    wsem = [ws0, ws1, ws2, ws3, ws4]
    n_ch = idx_all.shape[0]
    sid = lax.axis_index("s")
    wid = sid * NC + lax.axis_index("c")
    row_base = wid * n_ch

    seg = TAB_PAD // NS
    pltpu.sync_copy(tab_hbm.at[pl.ds(sid * seg, seg)],
                    tab_sp.at[pl.ds(sid * seg, seg)])

    pltpu.sync_copy(idx_hbm.at[wid], idx_all)

    plsc.subcore_barrier()

    def g_start(g, b):
        pltpu.make_async_copy(tab_sp.at[idx_all.at[g]], rows[b], gsem[b]).start()

    def g_wait(g, b):
        pltpu.make_async_copy(tab_sp.at[idx_all.at[g]], rows[b], gsem[b]).wait()

    def w_start(g, b):
        base = pl.multiple_of((row_base + g) * CH, CH)
        pltpu.make_async_copy(rows[b], out_hbm.at[pl.ds(base, CH)], wsem[b]).start()

    def w_wait(g, b):
        base = pl.multiple_of((row_base + g) * CH, CH)
        pltpu.make_async_copy(rows[b], out_hbm.at[pl.ds(base, CH)], wsem[b]).wait()

    for i in range(NBUF):
        g_start(i, i)
        if i >= W:
            g_wait(i - W, i - W)
            w_start(i - W, i - W)

    def outer(q, c):
        g0 = q * NBUF
        for b in range(NBUF):
            g = g0 + b
            w_wait(g - NBUF, b)
            g_start(g, b)
            b2 = (b - W) % NBUF
            g_wait(g - W, b2)
            w_start(g - W, b2)
        return c

    lax.fori_loop(1, n_ch // NBUF, outer, 0)

    for i in range(W):
        g = n_ch - W + i
        b = g % NBUF
        g_wait(g, b)
        w_start(g, b)
    for b in range(NBUF):
        w_wait(n_ch - NBUF + b, b)


def kernel(x, tables):
    B, T, C = x.shape
    n = B * T * C
    n_ch = n // CH // NW
    bias = (jnp.arange(C, dtype=jnp.int32) * VOCAB_P1).reshape(1, 1, C)
    flat_idx = (x.astype(jnp.int32) + bias).reshape(NW, n_ch, CH)
    flat_tab = tables.reshape(-1, EMBED)
    flat_tab = jnp.concatenate(
        [flat_tab, jnp.zeros((TAB_PAD - flat_tab.shape[0], EMBED), jnp.float32)])

    k = pl.kernel(
        _body,
        out_type=jax.ShapeDtypeStruct((n, EMBED), jnp.float32),
        mesh=plsc.VectorSubcoreMesh(core_axis_name="c", subcore_axis_name="s"),
        scratch_types=(
            [pltpu.VMEM_SHARED((TAB_PAD, EMBED), jnp.float32)]
            + [pltpu.VMEM((n_ch, CH), jnp.int32)]
            + [pltpu.VMEM((CH, EMBED), jnp.float32)] * NBUF
            + [pltpu.SemaphoreType.DMA] * (2 * NBUF)
        ),
    )
    out = k(flat_tab, flat_idx)
    return out.reshape(B, T, C, EMBED)


# speedup vs baseline: 1.0374x; 1.0063x over previous
import jax
import jax.numpy as jnp
from jax import lax
from jax.experimental import pallas as pl
from jax.experimental.pallas import tpu as pltpu
from jax.experimental.pallas import tpu_sc as plsc

NUM_CHANNELS = 8
VOCAB_P1 = 1001
EMBED = 128

NC, NS = 2, 16
NW = NC * NS

CH = 64
NBUF = 5
W = 3

TAB_PAD = 8064


def _body(tab_hbm, idx_hbm, out_hbm, tab_sp, idx_all,
          r0, r1, r2, r3, r4, gs0, gs1, gs2, gs3, gs4, ws0, ws1, ws2, ws3, ws4):
    rows = [r0, r1, r2, r3, r4]
    gsem = [gs0, gs1, gs2, gs3, gs4]
    wsem = [ws0, ws1, ws2, ws3, ws4]
    n_ch = idx_all.shape[0] // CH
    sid = lax.axis_index("s")
    wid = sid * NC + lax.axis_index("c")
    row_base = wid * n_ch

    seg = TAB_PAD // NS
    pltpu.sync_copy(tab_hbm.at[pl.ds(sid * seg, seg)],
                    tab_sp.at[pl.ds(sid * seg, seg)])

    pltpu.sync_copy(idx_hbm.at[pl.ds(wid * n_ch * CH, n_ch * CH)], idx_all)

    plsc.subcore_barrier()

    def g_start(g, b):
        pltpu.make_async_copy(tab_sp.at[idx_all.at[pl.ds(g * CH, CH)]],
                              rows[b], gsem[b]).start()

    def g_wait(g, b):
        pltpu.make_async_copy(tab_sp.at[idx_all.at[pl.ds(g * CH, CH)]],
                              rows[b], gsem[b]).wait()

    def w_start(g, b):
        base = pl.multiple_of((row_base + g) * CH, CH)
        pltpu.make_async_copy(rows[b], out_hbm.at[pl.ds(base, CH)], wsem[b]).start()

    def w_wait(g, b):
        base = pl.multiple_of((row_base + g) * CH, CH)
        pltpu.make_async_copy(rows[b], out_hbm.at[pl.ds(base, CH)], wsem[b]).wait()

    for i in range(NBUF):
        g_start(i, i)
        if i >= W:
            g_wait(i - W, i - W)
            w_start(i - W, i - W)

    def outer(q, c):
        g0 = q * NBUF
        for b in range(NBUF):
            g = g0 + b
            w_wait(g - NBUF, b)
            g_start(g, b)
            b2 = (b - W) % NBUF
            g_wait(g - W, b2)
            w_start(g - W, b2)
        return c

    lax.fori_loop(1, n_ch // NBUF, outer, 0)

    for i in range(W):
        g = n_ch - W + i
        b = g % NBUF
        g_wait(g, b)
        w_start(g, b)
    for b in range(NBUF):
        w_wait(n_ch - NBUF + b, b)


def kernel(x, tables):
    B, T, C = x.shape
    n = B * T * C
    n_ch = n // CH // NW
    bias = (jnp.arange(C, dtype=jnp.int32) * VOCAB_P1).reshape(1, 1, C)
    flat_idx = (x.astype(jnp.int32) + bias).reshape(n)
    flat_tab = tables.reshape(-1, EMBED)
    flat_tab = jnp.concatenate(
        [flat_tab, jnp.zeros((TAB_PAD - flat_tab.shape[0], EMBED), jnp.float32)])

    k = pl.kernel(
        _body,
        out_type=jax.ShapeDtypeStruct((n, EMBED), jnp.float32),
        mesh=plsc.VectorSubcoreMesh(core_axis_name="c", subcore_axis_name="s"),
        scratch_types=(
            [pltpu.VMEM_SHARED((TAB_PAD, EMBED), jnp.float32)]
            + [pltpu.VMEM((n // NW,), jnp.int32)]
            + [pltpu.VMEM((CH, EMBED), jnp.float32)] * NBUF
            + [pltpu.SemaphoreType.DMA] * (2 * NBUF)
        ),
    )
    out = k(flat_tab, flat_idx)
    return out.reshape(B, T, C, EMBED)


# speedup vs baseline: 1.0391x; 1.0016x over previous
import jax
import jax.numpy as jnp
from jax import lax
from jax.experimental import pallas as pl
from jax.experimental.pallas import tpu as pltpu
from jax.experimental.pallas import tpu_sc as plsc

NUM_CHANNELS = 8
VOCAB_P1 = 1001
EMBED = 128

NC, NS = 2, 16
NW = NC * NS

CH = 64
NBUF = 5
W = 2

TAB_PAD = 8064


def _body(tab_hbm, idx_hbm, out_hbm, tab_sp, idx_all,
          r0, r1, r2, r3, r4, gs0, gs1, gs2, gs3, gs4, ws0, ws1, ws2, ws3, ws4):
    rows = [r0, r1, r2, r3, r4]
    gsem = [gs0, gs1, gs2, gs3, gs4]
    wsem = [ws0, ws1, ws2, ws3, ws4]
    n_ch = idx_all.shape[0] // CH
    sid = lax.axis_index("s")
    wid = sid * NC + lax.axis_index("c")
    row_base = wid * n_ch

    seg = TAB_PAD // NS
    pltpu.sync_copy(tab_hbm.at[pl.ds(sid * seg, seg)],
                    tab_sp.at[pl.ds(sid * seg, seg)])

    pltpu.sync_copy(idx_hbm.at[pl.ds(wid * n_ch * CH, n_ch * CH)], idx_all)

    plsc.subcore_barrier()

    def g_start(g, b):
        pltpu.make_async_copy(tab_sp.at[idx_all.at[pl.ds(g * CH, CH)]],
                              rows[b], gsem[b]).start()

    def g_wait(g, b):
        pltpu.make_async_copy(tab_sp.at[idx_all.at[pl.ds(g * CH, CH)]],
                              rows[b], gsem[b]).wait()

    def w_start(g, b):
        base = pl.multiple_of((row_base + g) * CH, CH)
        pltpu.make_async_copy(rows[b], out_hbm.at[pl.ds(base, CH)], wsem[b]).start()

    def w_wait(g, b):
        base = pl.multiple_of((row_base + g) * CH, CH)
        pltpu.make_async_copy(rows[b], out_hbm.at[pl.ds(base, CH)], wsem[b]).wait()

    for i in range(NBUF):
        g_start(i, i)
        if i >= W:
            g_wait(i - W, i - W)
            w_start(i - W, i - W)

    def outer(q, c):
        g0 = q * NBUF
        for b in range(NBUF):
            g = g0 + b
            w_wait(g - NBUF, b)
            g_start(g, b)
            b2 = (b - W) % NBUF
            g_wait(g - W, b2)
            w_start(g - W, b2)
        return c

    lax.fori_loop(1, n_ch // NBUF, outer, 0)

    for i in range(W):
        g = n_ch - W + i
        b = g % NBUF
        g_wait(g, b)
        w_start(g, b)
    for b in range(NBUF):
        w_wait(n_ch - NBUF + b, b)


def kernel(x, tables):
    B, T, C = x.shape
    n = B * T * C
    n_ch = n // CH // NW
    bias = (jnp.arange(C, dtype=jnp.int32) * VOCAB_P1).reshape(1, 1, C)
    flat_idx = (x.astype(jnp.int32) + bias).reshape(n)
    flat_tab = tables.reshape(-1, EMBED)
    flat_tab = jnp.concatenate(
        [flat_tab, jnp.zeros((TAB_PAD - flat_tab.shape[0], EMBED), jnp.float32)])

    k = pl.kernel(
        _body,
        out_type=jax.ShapeDtypeStruct((n, EMBED), jnp.float32),
        mesh=plsc.VectorSubcoreMesh(core_axis_name="c", subcore_axis_name="s"),
        scratch_types=(
            [pltpu.VMEM_SHARED((TAB_PAD, EMBED), jnp.float32)]
            + [pltpu.VMEM((n // NW,), jnp.int32)]
            + [pltpu.VMEM((CH, EMBED), jnp.float32)] * NBUF
            + [pltpu.SemaphoreType.DMA] * (2 * NBUF)
        ),
    )
    out = k(flat_tab, flat_idx)
    return out.reshape(B, T, C, EMBED)
